# Initial kernel scaffold; baseline (speedup 1.0000x reference)
import jax
import jax.numpy as jnp
from jax.experimental import pallas as pl


def kernel(adjacency_matrix, weight_matrix, data, W, b):
    raise NotImplementedError("write your pallas kernel here")


# speedup vs baseline: 1.0303x; 1.0303x over previous
import jax
import jax.numpy as jnp
from jax.experimental import pallas as pl
from jax.experimental.pallas import tpu as pltpu

BM = 200


def _wgcn_body(a_ref, wm_ref, x_ref, w_ref, b_ref, o_ref):
    aw = a_ref[...] * wm_ref[...]
    agg = jnp.dot(aw, x_ref[...], preferred_element_type=jnp.float32)
    proj = jnp.dot(agg, w_ref[...], preferred_element_type=jnp.float32)
    o_ref[...] = jnp.maximum(proj + b_ref[...], 0.0)


def kernel(adjacency_matrix, weight_matrix, data, W, b):
    n, c = data.shape
    f = W.shape[1]
    bm = BM if n % BM == 0 else n
    ni = n // bm
    b2 = b.reshape(1, f).astype(jnp.float32)

    return pl.pallas_call(
        _wgcn_body,
        grid=(ni,),
        in_specs=[
            pl.BlockSpec((bm, n), lambda i: (i, 0)),
            pl.BlockSpec((bm, n), lambda i: (i, 0)),
            pl.BlockSpec((n, c), lambda i: (0, 0)),
            pl.BlockSpec((c, f), lambda i: (0, 0)),
            pl.BlockSpec((1, f), lambda i: (0, 0)),
        ],
        out_specs=pl.BlockSpec((bm, f), lambda i: (i, 0)),
        out_shape=jax.ShapeDtypeStruct((n, f), jnp.float32),
        compiler_params=pltpu.CompilerParams(
            dimension_semantics=("parallel",),
        ),
    )(adjacency_matrix, weight_matrix, data, W, b2)
